# baseline (speedup 1.0000x reference)
import jax
import jax.numpy as jnp
from jax.experimental import pallas as pl
from jax.experimental.pallas import tpu as pltpu

SEQ_BLK = 1024
FEAT_BLK = 2048
GROUP = 8


def _group_scan(v):
    s = v.shape[0]
    shift = 1
    while shift < s:
        pad = jnp.zeros((shift, v.shape[1]), v.dtype)
        v = v + jnp.concatenate([pad, v[:-shift]], axis=0)
        shift *= 2
    return v


def _body(x_ref, o_ref, carry_ref):
    sb = pl.program_id(2)

    @pl.when(sb == 0)
    def _():
        carry_ref[...] = jnp.zeros_like(carry_ref)

    def step(g, carry):
        v = x_ref[0, pl.ds(g * GROUP, GROUP), :]
        v = _group_scan(v) + carry
        o_ref[0, pl.ds(g * GROUP, GROUP), :] = v
        return v[GROUP - 1:GROUP, :]

    carry = jax.lax.fori_loop(0, SEQ_BLK // GROUP, step, carry_ref[...],
                              unroll=8)
    carry_ref[...] = carry


def kernel(x, dim, dtype):
    b, s, f = x.shape
    grid = (b, f // FEAT_BLK, s // SEQ_BLK)
    out = pl.pallas_call(
        _body,
        grid=grid,
        in_specs=[pl.BlockSpec((1, SEQ_BLK, FEAT_BLK),
                               lambda b_, f_, s_: (b_, s_, f_))],
        out_specs=pl.BlockSpec((1, SEQ_BLK, FEAT_BLK),
                               lambda b_, f_, s_: (b_, s_, f_)),
        out_shape=jax.ShapeDtypeStruct(x.shape, x.dtype),
        scratch_shapes=[pltpu.VMEM((1, FEAT_BLK), x.dtype)],
        compiler_params=pltpu.CompilerParams(
            dimension_semantics=("parallel", "parallel", "arbitrary"),
        ),
    )(x)
    return out
